# initial kernel scaffold (unmeasured)
import jax
import jax.numpy as jnp
from jax import lax
from jax.experimental import pallas as pl
from jax.experimental.pallas import tpu as pltpu

N_DEV = 4


def kernel(x, w_mat):
    m_per, k = x.shape
    _, n = w_mat.shape
    n_per = n // N_DEV

    def body(x_ref, w_ref, out_ref, y_ref, amax_send, amax_recv,
             send_sems, recv_sems, am_send_sems, am_recv_sems):
        my = lax.axis_index("i")

        barrier_sem = pltpu.get_barrier_semaphore()
        for d in range(1, N_DEV):
            pl.semaphore_signal(
                barrier_sem, inc=1,
                device_id=((my + d) % N_DEV,),
                device_id_type=pl.DeviceIdType.MESH,
            )
        pl.semaphore_wait(barrier_sem, N_DEV - 1)

        descs = []
        amax = jnp.float32(0.0)
        for d in range(1, N_DEV):
            peer = (my + d) % N_DEV
            blk = jnp.dot(
                x_ref[...],
                w_ref[:, pl.ds(peer * n_per, n_per)],
                preferred_element_type=jnp.float32,
            )
            y_ref[d - 1] = blk
            amax = jnp.maximum(amax, jnp.max(jnp.abs(blk)))
            rdma = pltpu.make_async_remote_copy(
                src_ref=y_ref.at[d - 1],
                dst_ref=out_ref.at[pl.ds(my * m_per, m_per), :],
                send_sem=send_sems.at[d - 1],
                recv_sem=recv_sems.at[d - 1],
                device_id=(peer,),
                device_id_type=pl.DeviceIdType.MESH,
            )
            rdma.start()
            descs.append(rdma)

        blk = jnp.dot(
            x_ref[...],
            w_ref[:, pl.ds(my * n_per, n_per)],
            preferred_element_type=jnp.float32,
        )
        out_ref[pl.ds(my * m_per, m_per), :] = blk
        amax = jnp.maximum(amax, jnp.max(jnp.abs(blk)))

        amax_send[...] = jnp.full(amax_send.shape, amax, jnp.float32)
        for d in range(1, N_DEV):
            peer = (my + d) % N_DEV
            rdma = pltpu.make_async_remote_copy(
                src_ref=amax_send,
                dst_ref=amax_recv.at[d - 1],
                send_sem=am_send_sems.at[d - 1],
                recv_sem=am_recv_sems.at[d - 1],
                device_id=(peer,),
                device_id_type=pl.DeviceIdType.MESH,
            )
            rdma.start()
            descs.append(rdma)

        for rdma in descs:
            rdma.wait_send()
        for rdma in descs:
            rdma.wait_recv()

        gmax = jnp.maximum(amax, jnp.max(amax_recv[...]))
        scale = gmax / 448.0
        q = (out_ref[...] / scale).astype(jnp.float8_e4m3fn)
        out_ref[...] = q.astype(jnp.float32) * scale

    grid_spec = pltpu.PrefetchScalarGridSpec(
        num_scalar_prefetch=0,
        in_specs=[
            pl.BlockSpec(memory_space=pltpu.VMEM),
            pl.BlockSpec(memory_space=pltpu.VMEM),
        ],
        out_specs=pl.BlockSpec(memory_space=pltpu.VMEM),
        scratch_shapes=[
            pltpu.VMEM((N_DEV - 1, m_per, n_per), jnp.float32),
            pltpu.VMEM((8, 128), jnp.float32),
            pltpu.VMEM((N_DEV - 1, 8, 128), jnp.float32),
            pltpu.SemaphoreType.DMA((N_DEV - 1,)),
            pltpu.SemaphoreType.DMA((N_DEV - 1,)),
            pltpu.SemaphoreType.DMA((N_DEV - 1,)),
            pltpu.SemaphoreType.DMA((N_DEV - 1,)),
        ],
    )
    return pl.pallas_call(
        body,
        out_shape=jax.ShapeDtypeStruct((N_DEV * m_per, n_per), jnp.float32),
        grid_spec=grid_spec,
        compiler_params=pltpu.CompilerParams(collective_id=0),
    )(x, w_mat)


# baseline (device time: 80629 ns/iter reference)
import jax
import jax.numpy as jnp
from jax import lax
from jax.experimental import pallas as pl
from jax.experimental.pallas import tpu as pltpu

N_DEV = 4


def kernel(x, w_mat):
    m_per, k = x.shape
    _, n = w_mat.shape
    n_per = n // N_DEV

    def body(x_ref, w_ref, out_ref, w_buf, y_ref, amax_send, amax_recv,
             w_sems, send_sems, recv_sems, am_send_sems, am_recv_sems):
        my = lax.axis_index("i")

        barrier_sem = pltpu.get_barrier_semaphore()
        for d in range(1, N_DEV):
            pl.semaphore_signal(
                barrier_sem, inc=1,
                device_id=((my + d) % N_DEV,),
                device_id_type=pl.DeviceIdType.MESH,
            )
        pl.semaphore_wait(barrier_sem, N_DEV - 1)

        def col_of(t):
            return ((my + 1 + t) % N_DEV) * n_per

        def w_fetch(t):
            cp = pltpu.make_async_copy(
                w_ref.at[:, pl.ds(col_of(t), n_per)],
                w_buf.at[t % 2],
                w_sems.at[t % 2],
            )
            cp.start()
            return cp

        fetches = [w_fetch(0)]
        descs = []
        amax = jnp.float32(0.0)
        for t in range(N_DEV):
            if t + 1 < N_DEV:
                fetches.append(w_fetch(t + 1))
            fetches[t].wait()
            blk = jnp.dot(
                x_ref[...], w_buf[t % 2],
                preferred_element_type=jnp.float32,
            )
            amax = jnp.maximum(amax, jnp.max(jnp.abs(blk)))
            if t + 1 < N_DEV:
                y_ref[t] = blk
                rdma = pltpu.make_async_remote_copy(
                    src_ref=y_ref.at[t],
                    dst_ref=out_ref.at[pl.ds(my * m_per, m_per), :],
                    send_sem=send_sems.at[t],
                    recv_sem=recv_sems.at[t],
                    device_id=((my + 1 + t) % N_DEV,),
                    device_id_type=pl.DeviceIdType.MESH,
                )
                rdma.start()
                descs.append(rdma)
            else:
                out_ref[pl.ds(my * m_per, m_per), :] = blk

        amax_send[...] = jnp.full(amax_send.shape, amax, jnp.float32)
        for d in range(1, N_DEV):
            rdma = pltpu.make_async_remote_copy(
                src_ref=amax_send,
                dst_ref=amax_recv.at[d - 1],
                send_sem=am_send_sems.at[d - 1],
                recv_sem=am_recv_sems.at[d - 1],
                device_id=((my + d) % N_DEV,),
                device_id_type=pl.DeviceIdType.MESH,
            )
            rdma.start()
            descs.append(rdma)

        for rdma in descs:
            rdma.wait_send()
        for rdma in descs:
            rdma.wait_recv()

        gmax = jnp.maximum(amax, jnp.max(amax_recv[...]))
        scale = gmax / 448.0
        q = (out_ref[...] / scale).astype(jnp.float8_e4m3fn)
        out_ref[...] = q.astype(jnp.float32) * scale

    grid_spec = pltpu.PrefetchScalarGridSpec(
        num_scalar_prefetch=0,
        in_specs=[
            pl.BlockSpec(memory_space=pltpu.VMEM),
            pl.BlockSpec(memory_space=pltpu.MemorySpace.HBM),
        ],
        out_specs=pl.BlockSpec(memory_space=pltpu.VMEM),
        scratch_shapes=[
            pltpu.VMEM((2, k, n_per), jnp.float32),
            pltpu.VMEM((N_DEV - 1, m_per, n_per), jnp.float32),
            pltpu.VMEM((8, 128), jnp.float32),
            pltpu.VMEM((N_DEV - 1, 8, 128), jnp.float32),
            pltpu.SemaphoreType.DMA((2,)),
            pltpu.SemaphoreType.DMA((N_DEV - 1,)),
            pltpu.SemaphoreType.DMA((N_DEV - 1,)),
            pltpu.SemaphoreType.DMA((N_DEV - 1,)),
            pltpu.SemaphoreType.DMA((N_DEV - 1,)),
        ],
    )
    return pl.pallas_call(
        body,
        out_shape=jax.ShapeDtypeStruct((N_DEV * m_per, n_per), jnp.float32),
        grid_spec=grid_spec,
        compiler_params=pltpu.CompilerParams(
            collective_id=0, vmem_limit_bytes=100 * 1024 * 1024
        ),
    )(x, w_mat)


# device time: 58241 ns/iter; 1.3844x vs baseline; 1.3844x over previous
import jax
import jax.numpy as jnp
from jax import lax
from jax.experimental import pallas as pl
from jax.experimental.pallas import tpu as pltpu

N_DEV = 4


def kernel(x, w_mat):
    m_per, k = x.shape
    _, n = w_mat.shape
    n_per = n // N_DEV

    def body(x_ref, w_ref, out_ref, w_buf, y_ref, q_ref, qin_ref,
             amax_send, amax_recv, w_sems, send_sems, recv_sems,
             am_send_sems, am_recv_sems):
        my = lax.axis_index("i")

        barrier_sem = pltpu.get_barrier_semaphore()
        for d in range(1, N_DEV):
            pl.semaphore_signal(
                barrier_sem, inc=1,
                device_id=((my + d) % N_DEV,),
                device_id_type=pl.DeviceIdType.MESH,
            )
        pl.semaphore_wait(barrier_sem, N_DEV - 1)

        def col_of(t):
            return ((my + 1 + t) % N_DEV) * n_per

        def w_fetch(t):
            cp = pltpu.make_async_copy(
                w_ref.at[:, pl.ds(col_of(t), n_per)],
                w_buf.at[t % 2],
                w_sems.at[t % 2],
            )
            cp.start()
            return cp

        fetches = [w_fetch(0)]
        descs = []
        amax = jnp.float32(0.0)
        for t in range(N_DEV):
            if t + 1 < N_DEV:
                fetches.append(w_fetch(t + 1))
            fetches[t].wait()
            blk = jnp.dot(
                x_ref[...], w_buf[t % 2],
                preferred_element_type=jnp.float32,
            )
            amax = jnp.maximum(amax, jnp.max(jnp.abs(blk)))
            if t + 1 < N_DEV:
                y_ref[t] = blk
            else:
                out_ref[pl.ds(my * m_per, m_per), :] = blk

        amax_send[...] = jnp.full(amax_send.shape, amax, jnp.float32)
        for d in range(1, N_DEV):
            rdma = pltpu.make_async_remote_copy(
                src_ref=amax_send,
                dst_ref=amax_recv.at[d - 1],
                send_sem=am_send_sems.at[d - 1],
                recv_sem=am_recv_sems.at[d - 1],
                device_id=((my + d) % N_DEV,),
                device_id_type=pl.DeviceIdType.MESH,
            )
            rdma.start()
            descs.append(rdma)
        for d in range(1, N_DEV):
            descs[d - 1].wait_recv()

        gmax = jnp.maximum(amax, jnp.max(amax_recv[...]))
        scale = gmax / 448.0
        inv = 448.0 / gmax

        recvs = []
        for t in range(N_DEV - 1):
            q_ref[t] = (y_ref[t] * inv).astype(jnp.float8_e4m3fn)
            rdma = pltpu.make_async_remote_copy(
                src_ref=q_ref.at[t],
                dst_ref=qin_ref.at[t],
                send_sem=send_sems.at[t],
                recv_sem=recv_sems.at[t],
                device_id=((my + 1 + t) % N_DEV,),
                device_id_type=pl.DeviceIdType.MESH,
            )
            rdma.start()
            recvs.append(rdma)
            descs.append(rdma)

        own = out_ref[pl.ds(my * m_per, m_per), :]
        out_ref[pl.ds(my * m_per, m_per), :] = (
            (own * inv).astype(jnp.float8_e4m3fn).astype(jnp.float32) * scale
        )

        for t in range(N_DEV - 1):
            recvs[t].wait_recv()
            src = (my - 1 - t) % N_DEV
            out_ref[pl.ds(src * m_per, m_per), :] = (
                qin_ref[t].astype(jnp.float32) * scale
            )

        for rdma in descs:
            rdma.wait_send()

    grid_spec = pltpu.PrefetchScalarGridSpec(
        num_scalar_prefetch=0,
        in_specs=[
            pl.BlockSpec(memory_space=pltpu.MemorySpace.VMEM),
            pl.BlockSpec(memory_space=pltpu.MemorySpace.HBM),
        ],
        out_specs=pl.BlockSpec(memory_space=pltpu.MemorySpace.VMEM),
        scratch_shapes=[
            pltpu.VMEM((2, k, n_per), jnp.float32),
            pltpu.VMEM((N_DEV - 1, m_per, n_per), jnp.float32),
            pltpu.VMEM((N_DEV - 1, m_per, n_per), jnp.float8_e4m3fn),
            pltpu.VMEM((N_DEV - 1, m_per, n_per), jnp.float8_e4m3fn),
            pltpu.VMEM((8, 128), jnp.float32),
            pltpu.VMEM((N_DEV - 1, 8, 128), jnp.float32),
            pltpu.SemaphoreType.DMA((2,)),
            pltpu.SemaphoreType.DMA((N_DEV - 1,)),
            pltpu.SemaphoreType.DMA((N_DEV - 1,)),
            pltpu.SemaphoreType.DMA((N_DEV - 1,)),
            pltpu.SemaphoreType.DMA((N_DEV - 1,)),
        ],
    )
    return pl.pallas_call(
        body,
        out_shape=jax.ShapeDtypeStruct((N_DEV * m_per, n_per), jnp.float32),
        grid_spec=grid_spec,
        compiler_params=pltpu.CompilerParams(
            collective_id=0, vmem_limit_bytes=100 * 1024 * 1024
        ),
    )(x, w_mat)


# device time: 55277 ns/iter; 1.4586x vs baseline; 1.0536x over previous
import jax
import jax.numpy as jnp
from jax import lax
from jax.experimental import pallas as pl
from jax.experimental.pallas import tpu as pltpu

N_DEV = 4
ORDER = (2, 1, 3)
I16MAX = 32767.0


def kernel(x, w_mat):
    m_per, k = x.shape
    _, n = w_mat.shape
    n_per = n // N_DEV

    def body(x_ref, w_ref, out_ref, w_buf, iout_ref, iin_ref,
             amax_send, amax_recv, w_sems, send_sems, recv_sems,
             am_send_sems, am_recv_sems):
        my = lax.axis_index("i")

        barrier_sem = pltpu.get_barrier_semaphore()
        for d in range(1, N_DEV):
            pl.semaphore_signal(
                barrier_sem, inc=1,
                device_id=((my + d) % N_DEV,),
                device_id_type=pl.DeviceIdType.MESH,
            )
        pl.semaphore_wait(barrier_sem, N_DEV - 1)

        offs = list(ORDER) + [0]

        def w_fetch(t):
            cp = pltpu.make_async_copy(
                w_ref.at[:, pl.ds(((my + offs[t]) % N_DEV) * n_per, n_per)],
                w_buf.at[t % 2],
                w_sems.at[t % 2],
            )
            cp.start()
            return cp

        fetches = [w_fetch(0)]
        descs = []
        amax = jnp.float32(0.0)
        blkmax = {}
        for t in range(N_DEV):
            if t + 1 < N_DEV:
                fetches.append(w_fetch(t + 1))
            fetches[t].wait()
            blk = jnp.dot(
                x_ref[...], w_buf[t % 2],
                preferred_element_type=jnp.float32,
            )
            if t + 1 < N_DEV:
                d = offs[t]
                bm = jnp.maximum(jnp.max(jnp.abs(blk)), jnp.float32(1e-30))
                blkmax[d] = bm
                amax = jnp.maximum(amax, bm)
                iout_ref[d - 1] = jnp.round(blk * (I16MAX / bm)).astype(
                    jnp.int16
                )
                rdma = pltpu.make_async_remote_copy(
                    src_ref=iout_ref.at[d - 1],
                    dst_ref=iin_ref.at[d - 1],
                    send_sem=send_sems.at[d - 1],
                    recv_sem=recv_sems.at[d - 1],
                    device_id=((my + d) % N_DEV,),
                    device_id_type=pl.DeviceIdType.MESH,
                )
                rdma.start()
                descs.append(rdma)
            else:
                amax = jnp.maximum(amax, jnp.max(jnp.abs(blk)))
                out_ref[pl.ds(my * m_per, m_per), :] = blk

        col = lax.broadcasted_iota(jnp.int32, amax_send.shape, 1)
        msg = jnp.full(amax_send.shape, amax, jnp.float32)
        for d in range(1, N_DEV):
            msg = jnp.where(col == d, blkmax[d], msg)
        amax_send[...] = msg
        am_descs = []
        for d in range(1, N_DEV):
            rdma = pltpu.make_async_remote_copy(
                src_ref=amax_send,
                dst_ref=amax_recv.at[d - 1],
                send_sem=am_send_sems.at[d - 1],
                recv_sem=am_recv_sems.at[d - 1],
                device_id=((my + d) % N_DEV,),
                device_id_type=pl.DeviceIdType.MESH,
            )
            rdma.start()
            am_descs.append(rdma)
        for am in am_descs:
            am.wait_recv()

        gmax = jnp.maximum(amax, jnp.max(amax_recv[...]))
        scale = gmax / 448.0
        inv = 448.0 / gmax

        own = out_ref[pl.ds(my * m_per, m_per), :]
        out_ref[pl.ds(my * m_per, m_per), :] = (
            (own * inv).astype(jnp.float8_e4m3fn).astype(jnp.float32) * scale
        )

        for t in range(N_DEV - 1):
            d = offs[t]
            descs[t].wait_recv()
            src = (my - d) % N_DEV
            bm = jnp.max(amax_recv[d - 1, :, d])
            val = iin_ref[d - 1].astype(jnp.float32) * (bm / I16MAX)
            out_ref[pl.ds(src * m_per, m_per), :] = (
                (val * inv).astype(jnp.float8_e4m3fn).astype(jnp.float32)
                * scale
            )

        for rdma in descs + am_descs:
            rdma.wait_send()

    grid_spec = pltpu.PrefetchScalarGridSpec(
        num_scalar_prefetch=0,
        in_specs=[
            pl.BlockSpec(memory_space=pltpu.MemorySpace.VMEM),
            pl.BlockSpec(memory_space=pltpu.MemorySpace.HBM),
        ],
        out_specs=pl.BlockSpec(memory_space=pltpu.MemorySpace.VMEM),
        scratch_shapes=[
            pltpu.VMEM((2, k, n_per), jnp.float32),
            pltpu.VMEM((N_DEV - 1, m_per, n_per), jnp.int16),
            pltpu.VMEM((N_DEV - 1, m_per, n_per), jnp.int16),
            pltpu.VMEM((8, 128), jnp.float32),
            pltpu.VMEM((N_DEV - 1, 8, 128), jnp.float32),
            pltpu.SemaphoreType.DMA((2,)),
            pltpu.SemaphoreType.DMA((N_DEV - 1,)),
            pltpu.SemaphoreType.DMA((N_DEV - 1,)),
            pltpu.SemaphoreType.DMA((N_DEV - 1,)),
            pltpu.SemaphoreType.DMA((N_DEV - 1,)),
        ],
    )
    return pl.pallas_call(
        body,
        out_shape=jax.ShapeDtypeStruct((N_DEV * m_per, n_per), jnp.float32),
        grid_spec=grid_spec,
        compiler_params=pltpu.CompilerParams(
            collective_id=0, vmem_limit_bytes=100 * 1024 * 1024
        ),
    )(x, w_mat)
